# speedup vs baseline: 2.9633x; 2.9633x over previous
import functools

import jax
import jax.numpy as jnp
from jax import lax
from jax.experimental import pallas as pl
from jax.experimental.pallas import tpu as pltpu
from jax.experimental.pallas import tpu_sc as plsc

_NC = 2
_NS = 16
_NW = _NC * _NS

_CHUNK = 128


@functools.partial(jax.jit, static_argnames=("b_per_w", "n_chunks"))
def _sc_gather(flat_idx, table, *, b_per_w, n_chunks):
    B = flat_idx.shape[0]
    D = table.shape[1]
    mesh = plsc.VectorSubcoreMesh(core_axis_name="c", subcore_axis_name="s")

    @functools.partial(
        pl.kernel,
        out_type=jax.ShapeDtypeStruct((B, D), jnp.float32),
        mesh=mesh,
        scratch_types=[
            pltpu.VMEM((b_per_w,), jnp.int32),
            pltpu.VMEM((2, _CHUNK, D), jnp.float32),
            pltpu.SemaphoreType.DMA,
        ],
    )
    def k(idx_hbm, table_hbm, out_hbm, idx_v, rows_v, gsem):
        wid = lax.axis_index("s") * _NC + lax.axis_index("c")
        base = wid * b_per_w
        pltpu.sync_copy(idx_hbm.at[pl.ds(base, b_per_w)], idx_v)

        def step(c, carry):
            buf = rows_v.at[c % 2]
            pltpu.async_copy(
                table_hbm.at[idx_v.at[pl.ds(c * _CHUNK, _CHUNK)]], buf, gsem
            ).wait()
            pltpu.sync_copy(buf, out_hbm.at[pl.ds(base + c * _CHUNK, _CHUNK)])
            return carry

        lax.fori_loop(0, n_chunks, step, 0)

    return k(flat_idx, table)


def kernel(x, table):
    B, H = x.shape
    D = table.shape[1]
    flat = x.reshape(B * H).astype(jnp.int32)
    total = B * H
    b_per_w = total // _NW
    n_chunks = b_per_w // _CHUNK
    out = _sc_gather(flat, table, b_per_w=b_per_w, n_chunks=n_chunks)
    return out.reshape(B, H, D)


# speedup vs baseline: 3.3400x; 1.1271x over previous
import functools

import jax
import jax.numpy as jnp
from jax import lax
from jax.experimental import pallas as pl
from jax.experimental.pallas import tpu as pltpu
from jax.experimental.pallas import tpu_sc as plsc

_NC = 2
_NS = 16
_NW = _NC * _NS

_CHUNK = 128


@functools.partial(jax.jit, static_argnames=("b_per_w", "n_chunks"))
def _sc_gather(flat_idx, table, *, b_per_w, n_chunks):
    B = flat_idx.shape[0]
    D = table.shape[1]
    mesh = plsc.VectorSubcoreMesh(core_axis_name="c", subcore_axis_name="s")

    nbuf = 4

    @functools.partial(
        pl.kernel,
        out_type=jax.ShapeDtypeStruct((B, D), jnp.float32),
        mesh=mesh,
        scratch_types=[
            pltpu.VMEM((b_per_w,), jnp.int32),
            pltpu.VMEM((nbuf, _CHUNK, D), jnp.float32),
            pltpu.SemaphoreType.DMA((nbuf,)),
            pltpu.SemaphoreType.DMA((nbuf,)),
        ],
    )
    def k(idx_hbm, table_hbm, out_hbm, idx_v, rows_v, gsem, osem):
        wid = lax.axis_index("s") * _NC + lax.axis_index("c")
        base = wid * b_per_w
        pltpu.sync_copy(idx_hbm.at[pl.ds(base, b_per_w)], idx_v)

        def gather(c, s):
            return pltpu.make_async_copy(
                table_hbm.at[idx_v.at[pl.ds(c * _CHUNK, _CHUNK)]],
                rows_v.at[s],
                gsem.at[s],
            )

        def copy_out(c, s):
            return pltpu.make_async_copy(
                rows_v.at[s],
                out_hbm.at[pl.ds(base + c * _CHUNK, _CHUNK)],
                osem.at[s],
            )

        gather(0, 0).start()
        gather(1, 1).start()

        n_steps = n_chunks + 2

        def step(p, carry):
            for b in range(nbuf):
                c = nbuf * p + b
                s_next = (b + 2) % nbuf

                @pl.when(jnp.logical_and(c >= 2, c - 2 < n_chunks - nbuf))
                def _():
                    copy_out(c - 2, s_next).wait()

                @pl.when(c + 2 < n_chunks)
                def _():
                    gather(c + 2, s_next).start()

                @pl.when(c < n_chunks)
                def _():
                    gather(c, b).wait()
                    copy_out(c, b).start()

            return carry

        lax.fori_loop(0, (n_steps + nbuf - 1) // nbuf, step, 0)

        for t in range(nbuf):
            c = n_chunks - nbuf + t
            copy_out(c, c % nbuf).wait()

    return k(flat_idx, table)


def kernel(x, table):
    B, H = x.shape
    D = table.shape[1]
    flat = x.reshape(B * H).astype(jnp.int32)
    total = B * H
    b_per_w = total // _NW
    n_chunks = b_per_w // _CHUNK
    out = _sc_gather(flat, table, b_per_w=b_per_w, n_chunks=n_chunks)
    return out.reshape(B, H, D)


# speedup vs baseline: 5.9010x; 1.7668x over previous
import functools

import jax
import jax.numpy as jnp
from jax import lax
from jax.experimental import pallas as pl
from jax.experimental.pallas import tpu as pltpu
from jax.experimental.pallas import tpu_sc as plsc

_NC = 2
_NS = 16
_NW = _NC * _NS


@functools.partial(jax.jit, static_argnames=("rows_per_w",))
def _sc_gather(x, table, *, rows_per_w):
    H = x.shape[1]
    D = table.shape[1]
    n_chunks = rows_per_w
    mesh = plsc.VectorSubcoreMesh(core_axis_name="c", subcore_axis_name="s")

    nbuf = 4

    @functools.partial(
        pl.kernel,
        out_type=jax.ShapeDtypeStruct((x.shape[0], H, D), jnp.float32),
        mesh=mesh,
        scratch_types=[
            pltpu.VMEM((rows_per_w, H), jnp.int32),
            pltpu.VMEM((nbuf, H, D), jnp.float32),
            pltpu.SemaphoreType.DMA((nbuf,)),
            pltpu.SemaphoreType.DMA((nbuf,)),
        ],
    )
    def k(x_hbm, table_hbm, out_hbm, idx_v, rows_v, gsem, osem):
        wid = lax.axis_index("s") * _NC + lax.axis_index("c")
        row0 = wid * rows_per_w
        pltpu.sync_copy(x_hbm.at[pl.ds(row0, rows_per_w)], idx_v)

        def gather(c, s):
            return pltpu.make_async_copy(
                table_hbm.at[idx_v.at[c]],
                rows_v.at[s],
                gsem.at[s],
            )

        def copy_out(c, s):
            return pltpu.make_async_copy(
                rows_v.at[s],
                out_hbm.at[row0 + c],
                osem.at[s],
            )

        gather(0, 0).start()
        gather(1, 1).start()

        n_steps = n_chunks + 2

        def step(p, carry):
            for b in range(nbuf):
                c = nbuf * p + b
                s_next = (b + 2) % nbuf

                @pl.when(jnp.logical_and(c >= 2, c - 2 < n_chunks - nbuf))
                def _():
                    copy_out(c - 2, s_next).wait()

                @pl.when(c + 2 < n_chunks)
                def _():
                    gather(c + 2, s_next).start()

                @pl.when(c < n_chunks)
                def _():
                    gather(c, b).wait()
                    copy_out(c, b).start()

            return carry

        lax.fori_loop(0, (n_steps + nbuf - 1) // nbuf, step, 0)

        for t in range(nbuf):
            c = n_chunks - nbuf + t
            copy_out(c, c % nbuf).wait()

    return k(x, table)


def kernel(x, table):
    B, H = x.shape
    rows_per_w = B // _NW
    return _sc_gather(x.astype(jnp.int32), table, rows_per_w=rows_per_w)


# speedup vs baseline: 5.9987x; 1.0165x over previous
import functools

import jax
import jax.numpy as jnp
from jax import lax
from jax.experimental import pallas as pl
from jax.experimental.pallas import tpu as pltpu
from jax.experimental.pallas import tpu_sc as plsc

_NC = 2
_NS = 16
_NW = _NC * _NS


@functools.partial(jax.jit, static_argnames=("rows_per_w",))
def _sc_gather(x, table, *, rows_per_w):
    H = x.shape[1]
    D = table.shape[1]
    mesh = plsc.VectorSubcoreMesh(core_axis_name="c", subcore_axis_name="s")

    nbuf = 4
    grp = 4
    n_chunks = rows_per_w // grp

    @functools.partial(
        pl.kernel,
        out_type=jax.ShapeDtypeStruct((x.shape[0], H, D), jnp.float32),
        mesh=mesh,
        scratch_types=[
            pltpu.VMEM((rows_per_w, H), jnp.int32),
            pltpu.VMEM((nbuf, grp, H, D), jnp.float32),
            pltpu.SemaphoreType.DMA((nbuf,)),
            pltpu.SemaphoreType.DMA((nbuf,)),
        ],
    )
    def k(x_hbm, table_hbm, out_hbm, idx_v, rows_v, gsem, osem):
        wid = lax.axis_index("s") * _NC + lax.axis_index("c")
        row0 = wid * rows_per_w
        pltpu.sync_copy(x_hbm.at[pl.ds(row0, rows_per_w)], idx_v)

        class _Group:
            def __init__(self, c, s):
                self.descs = [
                    pltpu.make_async_copy(
                        table_hbm.at[idx_v.at[c * grp + j]],
                        rows_v.at[s, j],
                        gsem.at[s],
                    )
                    for j in range(grp)
                ]

            def start(self):
                for d in self.descs:
                    d.start()

            def wait(self):
                for d in self.descs:
                    d.wait()

        gather = _Group

        def copy_out(c, s):
            return pltpu.make_async_copy(
                rows_v.at[s],
                out_hbm.at[pl.ds(row0 + c * grp, grp)],
                osem.at[s],
            )

        gather(0, 0).start()
        gather(1, 1).start()

        n_steps = n_chunks + 2

        def step(p, carry):
            for b in range(nbuf):
                c = nbuf * p + b
                s_next = (b + 2) % nbuf

                @pl.when(jnp.logical_and(c >= 2, c - 2 < n_chunks - nbuf))
                def _():
                    copy_out(c - 2, s_next).wait()

                @pl.when(c + 2 < n_chunks)
                def _():
                    gather(c + 2, s_next).start()

                @pl.when(c < n_chunks)
                def _():
                    gather(c, b).wait()
                    copy_out(c, b).start()

            return carry

        lax.fori_loop(0, (n_steps + nbuf - 1) // nbuf, step, 0)

        for t in range(nbuf):
            c = n_chunks - nbuf + t
            copy_out(c, c % nbuf).wait()

    return k(x, table)


def kernel(x, table):
    B, H = x.shape
    rows_per_w = B // _NW
    return _sc_gather(x.astype(jnp.int32), table, rows_per_w=rows_per_w)
